# baseline (speedup 1.0000x reference)
import functools

import jax
import jax.numpy as jnp
from jax.experimental import pallas as pl
from jax.experimental.pallas import tpu as pltpu


def _pick_block(n, candidates):
    for c in candidates:
        if n % c == 0:
            return c
    return n


def _prep_body(xa, xb, wp_a, wself_a, wsh_a, wc_a, b_a,
               wp_b, wself_b, wsh_b, wc_b, b_b,
               ya, yb, sa, sb):
    d_out = wsh_a.shape[0]
    f_a = jnp.dot(jnp.dot(wp_a[...], wsh_a[...]), wc_a[:d_out, :],
                  preferred_element_type=jnp.float32)
    g_a = jnp.dot(jnp.dot(wself_a[...], wsh_a[...]), wc_a[d_out:, :],
                  preferred_element_type=jnp.float32)
    f_b = jnp.dot(jnp.dot(wp_b[...], wsh_b[...]), wc_b[:d_out, :],
                  preferred_element_type=jnp.float32)
    g_b = jnp.dot(jnp.dot(wself_b[...], wsh_b[...]), wc_b[d_out:, :],
                  preferred_element_type=jnp.float32)
    ya[...] = jnp.dot(xb[...], f_a, preferred_element_type=jnp.float32)
    yb[...] = jnp.dot(xa[...], f_b, preferred_element_type=jnp.float32)
    sa[...] = jnp.dot(xa[...], g_a, preferred_element_type=jnp.float32) + b_a[...]
    sb[...] = jnp.dot(xb[...], g_b, preferred_element_type=jnp.float32) + b_b[...]


def _agg_body(adj_ab, adj_ba, ya, yb, sa, sb, out_a, out_b):
    out_a[...] = sa[...] + jnp.dot(adj_ab[...], ya[...],
                                   preferred_element_type=jnp.float32)
    out_b[...] = sb[...] + jnp.dot(adj_ba[...], yb[...],
                                   preferred_element_type=jnp.float32)


def kernel(x_a, x_b, adj_a_b, adj_b_a, W_proj_a_b, w_self_a, w_share_a,
           w_att_a, w_cat_a, bias_a, W_proj_b_a, w_self_b, w_share_b,
           w_att_b, w_cat_b, bias_b):
    n, d_in = x_a.shape
    d_out = w_share_a.shape[0]

    bm_prep = _pick_block(n, (1000, 500, 200, 8))
    rep = pl.BlockSpec((bm_prep, d_in), lambda i: (i, 0))
    wfull = lambda shape: pl.BlockSpec(shape, lambda i: (0,) * len(shape))
    yspec = pl.BlockSpec((bm_prep, d_out), lambda i: (i, 0))

    y_a, y_b, s_a, s_b = pl.pallas_call(
        _prep_body,
        grid=(n // bm_prep,),
        in_specs=[
            rep, rep,
            wfull((d_in, d_out)), wfull((d_in, d_out)),
            wfull((d_out, d_out)), wfull((2 * d_out, d_out)),
            wfull((1, d_out)),
            wfull((d_in, d_out)), wfull((d_in, d_out)),
            wfull((d_out, d_out)), wfull((2 * d_out, d_out)),
            wfull((1, d_out)),
        ],
        out_specs=[yspec, yspec, yspec, yspec],
        out_shape=[jax.ShapeDtypeStruct((n, d_out), jnp.float32)] * 4,
        compiler_params=pltpu.CompilerParams(
            dimension_semantics=("parallel",)),
    )(x_a, x_b, W_proj_a_b, w_self_a, w_share_a, w_cat_a, bias_a,
      W_proj_b_a, w_self_b, w_share_b, w_cat_b, bias_b)

    bm = _pick_block(n, (200, 40, 8))
    grid = (n // bm,)

    adj_spec = pl.BlockSpec((bm, n), lambda m: (m, 0))
    y_spec = pl.BlockSpec((n, d_out), lambda m: (0, 0))
    s_spec = pl.BlockSpec((bm, d_out), lambda m: (m, 0))
    o_spec = pl.BlockSpec((bm, d_out), lambda m: (m, 0))

    out_a, out_b = pl.pallas_call(
        _agg_body,
        grid=grid,
        in_specs=[adj_spec, adj_spec, y_spec, y_spec, s_spec, s_spec],
        out_specs=[o_spec, o_spec],
        out_shape=[jax.ShapeDtypeStruct((n, d_out), jnp.float32)] * 2,
        compiler_params=pltpu.CompilerParams(
            dimension_semantics=("arbitrary",)),
    )(adj_a_b, adj_b_a, y_a, y_b, s_a, s_b)

    return (out_a, out_b)


# speedup vs baseline: 1.0806x; 1.0806x over previous
import jax
import jax.numpy as jnp
from jax.experimental import pallas as pl
from jax.experimental.pallas import tpu as pltpu


def _pick_block(n, candidates):
    for c in candidates:
        if n % c == 0:
            return c
    return n


def _body(adj_ab, adj_ba, xa, xb, wp_a, wself_a, wsh_a, wc_a, b_a,
          wp_b, wself_b, wsh_b, wc_b, b_b, out_a, out_b, ya_s, yb_s):
    m = pl.program_id(0)
    bm = out_a.shape[0]
    d_out = wsh_a.shape[0]

    @pl.when(m == 0)
    def _compute_y():
        f_a = jnp.dot(jnp.dot(wp_a[...], wsh_a[...]), wc_a[:d_out, :],
                      preferred_element_type=jnp.float32)
        f_b = jnp.dot(jnp.dot(wp_b[...], wsh_b[...]), wc_b[:d_out, :],
                      preferred_element_type=jnp.float32)
        ya_s[...] = jnp.dot(xb[...], f_a, preferred_element_type=jnp.float32)
        yb_s[...] = jnp.dot(xa[...], f_b, preferred_element_type=jnp.float32)

    g_a = jnp.dot(jnp.dot(wself_a[...], wsh_a[...]), wc_a[d_out:, :],
                  preferred_element_type=jnp.float32)
    g_b = jnp.dot(jnp.dot(wself_b[...], wsh_b[...]), wc_b[d_out:, :],
                  preferred_element_type=jnp.float32)
    xa_blk = xa[pl.ds(m * bm, bm), :]
    xb_blk = xb[pl.ds(m * bm, bm), :]
    out_a[...] = (jnp.dot(adj_ab[...], ya_s[...],
                          preferred_element_type=jnp.float32)
                  + jnp.dot(xa_blk, g_a, preferred_element_type=jnp.float32)
                  + b_a[...])
    out_b[...] = (jnp.dot(adj_ba[...], yb_s[...],
                          preferred_element_type=jnp.float32)
                  + jnp.dot(xb_blk, g_b, preferred_element_type=jnp.float32)
                  + b_b[...])


def kernel(x_a, x_b, adj_a_b, adj_b_a, W_proj_a_b, w_self_a, w_share_a,
           w_att_a, w_cat_a, bias_a, W_proj_b_a, w_self_b, w_share_b,
           w_att_b, w_cat_b, bias_b):
    n, d_in = x_a.shape
    d_out = w_share_a.shape[0]

    bm = _pick_block(n, (200, 40, 8))
    grid = (n // bm,)

    adj_spec = pl.BlockSpec((bm, n), lambda m: (m, 0))
    full = lambda shape: pl.BlockSpec(shape, lambda m: (0,) * len(shape))
    o_spec = pl.BlockSpec((bm, d_out), lambda m: (m, 0))

    out_a, out_b = pl.pallas_call(
        _body,
        grid=grid,
        in_specs=[
            adj_spec, adj_spec,
            full((n, d_in)), full((n, d_in)),
            full((d_in, d_out)), full((d_in, d_out)),
            full((d_out, d_out)), full((2 * d_out, d_out)),
            full((1, d_out)),
            full((d_in, d_out)), full((d_in, d_out)),
            full((d_out, d_out)), full((2 * d_out, d_out)),
            full((1, d_out)),
        ],
        out_specs=[o_spec, o_spec],
        out_shape=[jax.ShapeDtypeStruct((n, d_out), jnp.float32)] * 2,
        scratch_shapes=[pltpu.VMEM((n, d_out), jnp.float32)] * 2,
        compiler_params=pltpu.CompilerParams(
            dimension_semantics=("arbitrary",)),
    )(adj_a_b, adj_b_a, x_a, x_b,
      W_proj_a_b, w_self_a, w_share_a, w_cat_a, bias_a,
      W_proj_b_a, w_self_b, w_share_b, w_cat_b, bias_b)

    return (out_a, out_b)
